# Initial kernel scaffold; baseline (speedup 1.0000x reference)
import jax
import jax.numpy as jnp
from jax.experimental import pallas as pl


def kernel(feature, edge_index, w):
    raise NotImplementedError("write your pallas kernel here")


# speedup vs baseline: 3.4240x; 3.4240x over previous
import functools

import numpy as np
import jax
import jax.numpy as jnp
from jax import lax
from jax.experimental import pallas as pl
from jax.experimental.pallas import tpu as pltpu
from jax.experimental.pallas import tpu_sc as plsc

N = 10000
E = 320000
D = 128
DROP = int(0.7 * E)

NC, NS, L = 2, 16, 16
NW = NC * NS
RPW = 80
R2D = NW * RPW
E_PAD = R2D * 128
PADE = E_PAD - E
RPS = R2D // NS
NPADH = 10240
NPT = NPADH // NS
NACC = 10240
NROWS_T = NACC // NS

_PAD_IDX = np.arange(PADE, dtype=np.int32) % N


def _rsqrt_vec(x):
    y = 1.0 / x
    for _ in range(26):
        y = y * (1.5 - 0.5 * x * y * y)
    return y


def _prep_body(src_hbm, dst_hbm, w_hbm, keep_hbm, wp_hbm,
               src_v, dst_v, w_v, keep_v, wp_v,
               onorm_v, inorm_v, tmp_v, zeros_v, ones_v,
               hout_s, hin_s):
    c = lax.axis_index("c")
    s = lax.axis_index("s")
    wid = s * NC + c

    pltpu.sync_copy(src_hbm.at[pl.ds(s * RPS, RPS)], src_v)
    pltpu.sync_copy(dst_hbm.at[pl.ds(s * RPS, RPS)], dst_v)

    zv = jnp.zeros((L,), jnp.float32)
    ov = jnp.ones((L,), jnp.float32)

    def _fill_zero(i, _):
        zeros_v[pl.ds(i * L, L)] = zv
        return 0
    lax.fori_loop(0, NPT // L, _fill_zero, 0)
    for k in range(128 // L):
        ones_v[pl.ds(k * L, L)] = ov

    pltpu.sync_copy(zeros_v, hout_s.at[pl.ds(s * NPT, NPT)])
    pltpu.sync_copy(zeros_v, hin_s.at[pl.ds(s * NPT, NPT)])
    plsc.subcore_barrier()

    def _hist(j, _):
        pltpu.sync_copy(ones_v, hout_s.at[src_v.at[j]], add=True)
        pltpu.sync_copy(ones_v, hin_s.at[dst_v.at[j]], add=True)
        return 0
    lax.fori_loop(0, RPS, _hist, 0)
    plsc.subcore_barrier()

    iota = lax.iota(jnp.int32, L)
    for hist in (hout_s, hin_s):
        pltpu.sync_copy(hist.at[pl.ds(s * NPT, NPT)], tmp_v)

        def _norm(i, _):
            hv = tmp_v[pl.ds(i * L, L)]
            nvec = s * NPT + i * L + iota
            padc = jnp.where(nvec < PADE, 1.0, 0.0).astype(jnp.float32)
            deg = jnp.maximum(hv - padc, 1.0)
            tmp_v[pl.ds(i * L, L)] = _rsqrt_vec(deg)
            return 0
        lax.fori_loop(0, NPT // L, _norm, 0)
        pltpu.sync_copy(tmp_v, hist.at[pl.ds(s * NPT, NPT)])
    plsc.subcore_barrier()

    pltpu.sync_copy(hout_s, onorm_v)
    pltpu.sync_copy(hin_s, inorm_v)

    r0 = wid * RPW
    loc = c * RPW
    pltpu.sync_copy(w_hbm.at[pl.ds(r0, RPW)], w_v)
    pltpu.sync_copy(keep_hbm.at[pl.ds(r0, RPW)], keep_v)

    def _wp(j, _):
        for k in range(128 // L):
            sl = pl.ds(k * L, L)
            sidx = src_v[loc + j, sl]
            didx = dst_v[loc + j, sl]
            on = plsc.load_gather(onorm_v, [sidx])
            inr = plsc.load_gather(inorm_v, [didx])
            wp_v[j, sl] = w_v[j, sl] * keep_v[j, sl] * on * inr
        return 0
    lax.fori_loop(0, RPW, _wp, 0)
    pltpu.sync_copy(wp_v, wp_hbm.at[pl.ds(r0, RPW)])


_prep = pl.kernel(
    _prep_body,
    out_type=jax.ShapeDtypeStruct((R2D, 128), jnp.float32),
    compiler_params=pltpu.CompilerParams(needs_layout_passes=False),
    mesh=plsc.VectorSubcoreMesh(core_axis_name="c", subcore_axis_name="s"),
    scratch_types=[
        pltpu.VMEM((RPS, 128), jnp.int32),
        pltpu.VMEM((RPS, 128), jnp.int32),
        pltpu.VMEM((RPW, 128), jnp.float32),
        pltpu.VMEM((RPW, 128), jnp.float32),
        pltpu.VMEM((RPW, 128), jnp.float32),
        pltpu.VMEM((NPADH,), jnp.float32),
        pltpu.VMEM((NPADH,), jnp.float32),
        pltpu.VMEM((NPT,), jnp.float32),
        pltpu.VMEM((NPT,), jnp.float32),
        pltpu.VMEM((128,), jnp.float32),
        pltpu.VMEM_SHARED((NPADH,), jnp.float32),
        pltpu.VMEM_SHARED((NPADH,), jnp.float32),
    ],
)


def _spmm_body(tbl_hbm, src_hbm, dst_hbm, wp_hbm, p_hbm,
               src_v, dst_v, wp_v, rowbuf, acc_s, gsem):
    c = lax.axis_index("c")
    s = lax.axis_index("s")
    wid = s * NC + c
    r0 = wid * RPW

    pltpu.sync_copy(src_hbm.at[pl.ds(r0, RPW)], src_v)
    pltpu.sync_copy(dst_hbm.at[pl.ds(r0, RPW)], dst_v)
    pltpu.sync_copy(wp_hbm.at[pl.ds(r0, RPW)], wp_v)

    zv = jnp.zeros((L,), jnp.float32)

    def _zero(i, _):
        for k in range(D // L):
            rowbuf[i, pl.ds(k * L, L)] = zv
        return 0
    lax.fori_loop(0, 128, _zero, 0)
    for q in range(NROWS_T // 128):
        pltpu.sync_copy(rowbuf, acc_s.at[pl.ds(s * NROWS_T + q * 128, 128)])
    plsc.subcore_barrier()

    def _chunk(j, _):
        pltpu.async_copy(tbl_hbm.at[src_v.at[j]], rowbuf, gsem).wait()

        def _scale(g, _):
            wv = wp_v[j, pl.ds(g * L, L)]
            for rr in range(L):
                r = g * L + rr
                wsc = wv[rr]
                for k in range(D // L):
                    sl = pl.ds(k * L, L)
                    rowbuf[r, sl] = rowbuf[r, sl] * wsc
            return 0
        lax.fori_loop(0, 128 // L, _scale, 0)
        pltpu.sync_copy(rowbuf, acc_s.at[dst_v.at[j]], add=True)
        return 0
    lax.fori_loop(0, RPW, _chunk, 0)
    plsc.subcore_barrier()

    pltpu.sync_copy(acc_s.at[pl.ds(s * NROWS_T, NROWS_T)],
                    p_hbm.at[c, pl.ds(s * NROWS_T, NROWS_T)])


_spmm = pl.kernel(
    _spmm_body,
    out_type=jax.ShapeDtypeStruct((NC, NACC, D), jnp.float32),
    compiler_params=pltpu.CompilerParams(needs_layout_passes=False),
    mesh=plsc.VectorSubcoreMesh(core_axis_name="c", subcore_axis_name="s"),
    scratch_types=[
        pltpu.VMEM((RPW, 128), jnp.int32),
        pltpu.VMEM((RPW, 128), jnp.int32),
        pltpu.VMEM((RPW, 128), jnp.float32),
        pltpu.VMEM((128, D), jnp.float32),
        pltpu.VMEM_SHARED((NACC, D), jnp.float32),
        pltpu.SemaphoreType.DMA,
    ],
)


def _sum2_body(p_ref, o_ref):
    o_ref[...] = p_ref[0] + p_ref[1]


_combine = pl.pallas_call(
    _sum2_body,
    grid=(10,),
    in_specs=[pl.BlockSpec((2, 1000, D), lambda i: (0, i, 0))],
    out_specs=pl.BlockSpec((1000, D), lambda i: (i, 0)),
    out_shape=jax.ShapeDtypeStruct((N, D), jnp.float32),
)


def _final_body(f_ref, h1_ref, p_ref, o_ref):
    o_ref[...] = (f_ref[...] + h1_ref[...] + p_ref[0] + p_ref[1]) * (1.0 / 3.0)


_final = pl.pallas_call(
    _final_body,
    grid=(10,),
    in_specs=[pl.BlockSpec((1000, D), lambda i: (i, 0)),
              pl.BlockSpec((1000, D), lambda i: (i, 0)),
              pl.BlockSpec((2, 1000, D), lambda i: (0, i, 0))],
    out_specs=pl.BlockSpec((1000, D), lambda i: (i, 0)),
    out_shape=jax.ShapeDtypeStruct((N, D), jnp.float32),
)


def kernel(feature, edge_index, w):
    ridx = jax.random.randint(jax.random.key(1), (DROP,), 0, E)
    keep = jnp.ones((E,), jnp.float32).at[ridx].set(0.0)
    keep2d = jnp.concatenate(
        [keep, jnp.zeros((PADE,), jnp.float32)]).reshape(R2D, 128)

    src = edge_index[0]
    dst = edge_index[1]
    pad_idx = jnp.asarray(_PAD_IDX)
    srcp = jnp.concatenate([src, pad_idx]).reshape(R2D, 128)
    dstp = jnp.concatenate([dst, pad_idx]).reshape(R2D, 128)
    wpad = jnp.concatenate([w, jnp.zeros((PADE,), w.dtype)]).reshape(R2D, 128)

    wprime = _prep(srcp, dstp, wpad, keep2d)
    p1 = _spmm(feature, srcp, dstp, wprime)
    h1 = _combine(p1)
    p2 = _spmm(h1, srcp, dstp, wprime)
    return _final(feature, h1, p2)
